# Initial kernel scaffold; baseline (speedup 1.0000x reference)
import jax
import jax.numpy as jnp
from jax.experimental import pallas as pl


def kernel(cat_data, num_data, emb, W1, b1, g1, bt1, W2, b2, g2, bt2, W3, b3):
    raise NotImplementedError("write your pallas kernel here")


# speedup vs baseline: 7.7129x; 7.7129x over previous
import functools

import jax
import jax.numpy as jnp
from jax import lax
from jax.experimental import pallas as pl
from jax.experimental.pallas import tpu as pltpu
from jax.experimental.pallas import tpu_sc as plsc

B = 16384
F = 26
V = 100000
D = 16
NUM = 96
H = 512
EPS = 1e-5

NC = 2
NS = 16
NW = NC * NS
R = B * F
SUB = 128
GROUP = 8
ROWS_PER_GROUP = SUB * GROUP
CHUNKS_PER_W = R // (NW * SUB)
OUTER_PER_W = CHUNKS_PER_W // GROUP


def _gather_body(table_hbm, idx_hbm, out_hbm, idx_v, rows_v, sem):
    wid = lax.axis_index("s") * NC + lax.axis_index("c")

    def outer(c, carry):
        chunk0 = wid * CHUNKS_PER_W + c * GROUP
        pltpu.sync_copy(idx_hbm.at[pl.ds(chunk0, GROUP)], idx_v)
        copies = []
        for j in range(GROUP):
            copies.append(
                pltpu.async_copy(
                    table_hbm.at[idx_v.at[j]],
                    rows_v.at[pl.ds(j * SUB, SUB)],
                    sem,
                )
            )
        for cp in copies:
            cp.wait()
        pltpu.sync_copy(rows_v, out_hbm.at[pl.ds(chunk0 * SUB, ROWS_PER_GROUP)])
        return carry

    lax.fori_loop(0, OUTER_PER_W, outer, 0)


_gather = pl.kernel(
    _gather_body,
    out_type=jax.ShapeDtypeStruct((R, D), jnp.float32),
    mesh=plsc.VectorSubcoreMesh(core_axis_name="c", subcore_axis_name="s"),
    scratch_types=[
        pltpu.VMEM((GROUP, SUB), jnp.int32),
        pltpu.VMEM((ROWS_PER_GROUP, D), jnp.float32),
        pltpu.SemaphoreType.DMA,
    ],
    compiler_params=pltpu.CompilerParams(use_tc_tiling_on_sc=False),
)


NBR = 1024


def _mlp_body(xg, xn, w1a, w1b, b1r, w2f, b2r, w3f, b3r, out):
    z1 = jnp.dot(xg[...], w1a[...], preferred_element_type=jnp.float32)
    z1 = z1 + jnp.dot(xn[...], w1b[...], preferred_element_type=jnp.float32)
    z1 = jnp.maximum(z1 + b1r[...], 0.0)
    z2 = jnp.dot(z1, w2f[...], preferred_element_type=jnp.float32)
    z2 = jnp.maximum(z2 + b2r[...], 0.0)
    out[...] = jnp.dot(z2, w3f[...], preferred_element_type=jnp.float32) + b3r[...]


_mlp = pl.pallas_call(
    _mlp_body,
    grid=(B // NBR,),
    in_specs=[
        pl.BlockSpec((NBR, F * D), lambda i: (i, 0)),
        pl.BlockSpec((NBR, NUM), lambda i: (i, 0)),
        pl.BlockSpec((F * D, H), lambda i: (0, 0)),
        pl.BlockSpec((NUM, H), lambda i: (0, 0)),
        pl.BlockSpec((1, H), lambda i: (0, 0)),
        pl.BlockSpec((H, H), lambda i: (0, 0)),
        pl.BlockSpec((1, H), lambda i: (0, 0)),
        pl.BlockSpec((H, 1), lambda i: (0, 0)),
        pl.BlockSpec((1, 1), lambda i: (0, 0)),
    ],
    out_specs=pl.BlockSpec((NBR, 1), lambda i: (i, 0)),
    out_shape=jax.ShapeDtypeStruct((B, 1), jnp.float32),
)


def kernel(cat_data, num_data, emb, W1, b1, g1, bt1, W2, b2, g2, bt2, W3, b3):
    idx = cat_data.astype(jnp.int32) + (jnp.arange(F, dtype=jnp.int32) * V)[None, :]
    idx2 = idx.reshape(R // SUB, SUB)
    table = emb.reshape(F * V, D)

    gathered = _gather(table, idx2).reshape(B, F * D)

    s1 = g1 * (1.0 / jnp.sqrt(1.0 + EPS))
    s2 = g2 * (1.0 / jnp.sqrt(1.0 + EPS))
    W1t = W1.T
    w1a = W1t[: F * D]
    w1b = W1t[F * D :]
    w2f = s1[:, None] * W2.T
    b2f = bt1 @ W2.T + b2
    w3f = s2[:, None] * W3.T
    b3f = bt2 @ W3.T + b3

    return _mlp(
        gathered,
        num_data,
        w1a,
        w1b,
        b1.reshape(1, H),
        w2f,
        b2f.reshape(1, H),
        w3f,
        b3f.reshape(1, 1),
    )


# speedup vs baseline: 47.8435x; 6.2031x over previous
import jax
import jax.numpy as jnp
from jax import lax
from jax.experimental import pallas as pl
from jax.experimental.pallas import tpu as pltpu
from jax.experimental.pallas import tpu_sc as plsc

B = 16384
F = 26
V = 100000
D = 16
NUM = 96
H = 512
EPS = 1e-5

NC = 2
NS = 16
NW = NC * NS
NPLANES = F * D
PLANES_PER_W = NPLANES // NW
IDX_CHUNK = 4096


def _gather_body(emb_t, cat_t, out_hbm, plane_v, idx_v, out_v):
    wid = lax.axis_index("s") * NC + lax.axis_index("c")
    for k in range(PLANES_PER_W):
        p = wid * PLANES_PER_W + k
        f = p // D
        d = p % D
        pltpu.sync_copy(emb_t.at[f, d], plane_v)
        for c in range(B // IDX_CHUNK):
            pltpu.sync_copy(cat_t.at[f, pl.ds(c * IDX_CHUNK, IDX_CHUNK)], idx_v)

            @plsc.parallel_loop(0, IDX_CHUNK, step=16, unroll=8)
            def _gather16(i, _c=c):
                vals = plsc.load_gather(plane_v, [idx_v[pl.ds(i, 16)]])
                out_v[pl.ds(_c * IDX_CHUNK + i, 16)] = vals

        pltpu.sync_copy(out_v, out_hbm.at[p])


_gather = pl.kernel(
    _gather_body,
    out_type=jax.ShapeDtypeStruct((NPLANES, B), jnp.float32),
    mesh=plsc.VectorSubcoreMesh(core_axis_name="c", subcore_axis_name="s"),
    scratch_types=[
        pltpu.VMEM((V,), jnp.float32),
        pltpu.VMEM((IDX_CHUNK,), jnp.int32),
        pltpu.VMEM((B,), jnp.float32),
    ],
    compiler_params=pltpu.CompilerParams(needs_layout_passes=False),
)


NBC = 2048


def _mlp_body(xg, xn, w1a, w1b, b1r, w2f, b2r, w3f, b3r, out):
    z1 = jnp.dot(w1a[...], xg[...], preferred_element_type=jnp.float32)
    z1 = z1 + jnp.dot(w1b[...], xn[...], preferred_element_type=jnp.float32)
    z1 = jnp.maximum(z1 + b1r[...], 0.0)
    z2 = jnp.dot(w2f[...], z1, preferred_element_type=jnp.float32)
    z2 = jnp.maximum(z2 + b2r[...], 0.0)
    out[...] = jnp.dot(w3f[...], z2, preferred_element_type=jnp.float32) + b3r[...]


_mlp = pl.pallas_call(
    _mlp_body,
    grid=(B // NBC,),
    in_specs=[
        pl.BlockSpec((NPLANES, NBC), lambda i: (0, i)),
        pl.BlockSpec((NUM, NBC), lambda i: (0, i)),
        pl.BlockSpec((H, NPLANES), lambda i: (0, 0)),
        pl.BlockSpec((H, NUM), lambda i: (0, 0)),
        pl.BlockSpec((H, 1), lambda i: (0, 0)),
        pl.BlockSpec((H, H), lambda i: (0, 0)),
        pl.BlockSpec((H, 1), lambda i: (0, 0)),
        pl.BlockSpec((1, H), lambda i: (0, 0)),
        pl.BlockSpec((1, 1), lambda i: (0, 0)),
    ],
    out_specs=pl.BlockSpec((1, NBC), lambda i: (0, i)),
    out_shape=jax.ShapeDtypeStruct((1, B), jnp.float32),
)


def kernel(cat_data, num_data, emb, W1, b1, g1, bt1, W2, b2, g2, bt2, W3, b3):
    emb_t = jnp.transpose(emb, (0, 2, 1))
    cat_t = cat_data.T
    xn_t = num_data.T

    xg_t = _gather(emb_t, cat_t)

    s1 = g1 * (1.0 / jnp.sqrt(1.0 + EPS))
    s2 = g2 * (1.0 / jnp.sqrt(1.0 + EPS))
    w1a = W1[:, : F * D]
    w1b = W1[:, F * D :]
    w2f = W2 * s1[None, :]
    b2f = W2 @ bt1 + b2
    w3f = W3 * s2[None, :]
    b3f = W3 @ bt2 + b3

    out_row = _mlp(
        xg_t,
        xn_t,
        w1a,
        w1b,
        b1.reshape(H, 1),
        w2f,
        b2f.reshape(H, 1),
        w3f,
        b3f.reshape(1, 1),
    )
    return out_row.reshape(B, 1)


# speedup vs baseline: 55.7183x; 1.1646x over previous
import jax
import jax.numpy as jnp
from jax import lax
from jax.experimental import pallas as pl
from jax.experimental.pallas import tpu as pltpu
from jax.experimental.pallas import tpu_sc as plsc

B = 16384
F = 26
V = 100000
D = 16
NUM = 96
H = 512
EPS = 1e-5

NC = 2
NS = 16
NW = NC * NS
NPLANES = F * D
PLANES_PER_W = NPLANES // NW


def _gather_body(emb_t, cat_t, out_hbm, plane_v, buf_v):
    wid = lax.axis_index("s") * NC + lax.axis_index("c")
    for k in range(PLANES_PER_W):
        p = wid * PLANES_PER_W + k
        f = p // D
        d = p % D
        pltpu.sync_copy(emb_t.at[f, d], plane_v)
        pltpu.sync_copy(cat_t.at[f], buf_v)

        @plsc.parallel_loop(0, B, step=16, unroll=8)
        def _gather16(i):
            idx16 = plsc.bitcast(buf_v[pl.ds(i, 16)], jnp.int32)
            buf_v[pl.ds(i, 16)] = plsc.load_gather(plane_v, [idx16])

        pltpu.sync_copy(buf_v, out_hbm.at[p])


_gather = pl.kernel(
    _gather_body,
    out_type=jax.ShapeDtypeStruct((NPLANES, B), jnp.float32),
    mesh=plsc.VectorSubcoreMesh(core_axis_name="c", subcore_axis_name="s"),
    scratch_types=[
        pltpu.VMEM((V,), jnp.float32),
        pltpu.VMEM((B,), jnp.float32),
    ],
    compiler_params=pltpu.CompilerParams(needs_layout_passes=False),
)


NBC = 2048


def _mlp_body(xg, xn, w1a, w1b, b1r, w2f, b2r, w3f, b3r, out):
    z1 = jnp.dot(w1a[...], xg[...], preferred_element_type=jnp.float32)
    z1 = z1 + jnp.dot(w1b[...], xn[...], preferred_element_type=jnp.float32)
    z1 = jnp.maximum(z1 + b1r[...], 0.0)
    z2 = jnp.dot(w2f[...], z1, preferred_element_type=jnp.float32)
    z2 = jnp.maximum(z2 + b2r[...], 0.0)
    out[...] = jnp.dot(w3f[...], z2, preferred_element_type=jnp.float32) + b3r[...]


_mlp = pl.pallas_call(
    _mlp_body,
    grid=(B // NBC,),
    in_specs=[
        pl.BlockSpec((NPLANES, NBC), lambda i: (0, i)),
        pl.BlockSpec((NUM, NBC), lambda i: (0, i)),
        pl.BlockSpec((H, NPLANES), lambda i: (0, 0)),
        pl.BlockSpec((H, NUM), lambda i: (0, 0)),
        pl.BlockSpec((H, 1), lambda i: (0, 0)),
        pl.BlockSpec((H, H), lambda i: (0, 0)),
        pl.BlockSpec((H, 1), lambda i: (0, 0)),
        pl.BlockSpec((1, H), lambda i: (0, 0)),
        pl.BlockSpec((1, 1), lambda i: (0, 0)),
    ],
    out_specs=pl.BlockSpec((1, NBC), lambda i: (0, i)),
    out_shape=jax.ShapeDtypeStruct((1, B), jnp.float32),
)


def kernel(cat_data, num_data, emb, W1, b1, g1, bt1, W2, b2, g2, bt2, W3, b3):
    emb_t = jnp.transpose(emb, (0, 2, 1))
    cat_t = lax.bitcast_convert_type(cat_data, jnp.float32).T
    xn_t = num_data.T

    xg_t = _gather(emb_t, cat_t)

    s1 = g1 * (1.0 / jnp.sqrt(1.0 + EPS))
    s2 = g2 * (1.0 / jnp.sqrt(1.0 + EPS))
    w1a = W1[:, : F * D]
    w1b = W1[:, F * D :]
    w2f = W2 * s1[None, :]
    b2f = W2 @ bt1 + b2
    w3f = W3 * s2[None, :]
    b3f = W3 @ bt2 + b3

    out_row = _mlp(
        xg_t,
        xn_t,
        w1a,
        w1b,
        b1.reshape(H, 1),
        w2f,
        b2f.reshape(H, 1),
        w3f,
        b3f.reshape(1, 1),
    )
    return out_row.reshape(B, 1)


# speedup vs baseline: 55.7490x; 1.0006x over previous
import jax
import jax.numpy as jnp
from jax import lax
from jax.experimental import pallas as pl
from jax.experimental.pallas import tpu as pltpu
from jax.experimental.pallas import tpu_sc as plsc

B = 16384
F = 26
V = 100000
D = 16
NUM = 96
H = 512
EPS = 1e-5

NC = 2
NS = 16
NW = NC * NS
NPLANES = F * D
PLANES_PER_W = NPLANES // NW


def _gather_body(emb_t, cat_t, out_hbm, plane_v, buf_v):
    wid = lax.axis_index("s") * NC + lax.axis_index("c")
    for k in range(PLANES_PER_W):
        p = wid * PLANES_PER_W + k
        f = p // D
        d = p % D
        pltpu.sync_copy(emb_t.at[f, d], plane_v)
        pltpu.sync_copy(cat_t.at[f], buf_v)

        @plsc.parallel_loop(0, B, step=16, unroll=8)
        def _gather16(i):
            idx16 = plsc.bitcast(buf_v[pl.ds(i, 16)], jnp.int32)
            buf_v[pl.ds(i, 16)] = plsc.load_gather(plane_v, [idx16])

        pltpu.sync_copy(buf_v, out_hbm.at[p])


_gather = pl.kernel(
    _gather_body,
    out_type=jax.ShapeDtypeStruct((NPLANES, B), jnp.float32),
    mesh=plsc.VectorSubcoreMesh(core_axis_name="c", subcore_axis_name="s"),
    scratch_types=[
        pltpu.VMEM((V,), jnp.float32),
        pltpu.VMEM((B,), jnp.float32),
    ],
    compiler_params=pltpu.CompilerParams(needs_layout_passes=False),
)


NBC = 2048


def _mlp_body(xg, xn, w1a, w1b, b1r, w2f, b2r, w3f, b3r, out):
    xg_b = xg[...].astype(jnp.bfloat16)
    xn_b = xn[...].astype(jnp.bfloat16)
    z1 = jnp.dot(w1a[...], xg_b, preferred_element_type=jnp.float32)
    z1 = z1 + jnp.dot(w1b[...], xn_b, preferred_element_type=jnp.float32)
    z1 = jnp.maximum(z1 + b1r[...], 0.0).astype(jnp.bfloat16)
    z2 = jnp.dot(w2f[...], z1, preferred_element_type=jnp.float32)
    z2 = jnp.maximum(z2 + b2r[...], 0.0).astype(jnp.bfloat16)
    out[...] = jnp.dot(w3f[...], z2, preferred_element_type=jnp.float32) + b3r[...]


_mlp = pl.pallas_call(
    _mlp_body,
    grid=(B // NBC,),
    in_specs=[
        pl.BlockSpec((NPLANES, NBC), lambda i: (0, i)),
        pl.BlockSpec((NUM, NBC), lambda i: (0, i)),
        pl.BlockSpec((H, NPLANES), lambda i: (0, 0)),
        pl.BlockSpec((H, NUM), lambda i: (0, 0)),
        pl.BlockSpec((H, 1), lambda i: (0, 0)),
        pl.BlockSpec((H, H), lambda i: (0, 0)),
        pl.BlockSpec((H, 1), lambda i: (0, 0)),
        pl.BlockSpec((1, H), lambda i: (0, 0)),
        pl.BlockSpec((1, 1), lambda i: (0, 0)),
    ],
    out_specs=pl.BlockSpec((1, NBC), lambda i: (0, i)),
    out_shape=jax.ShapeDtypeStruct((1, B), jnp.float32),
)


def kernel(cat_data, num_data, emb, W1, b1, g1, bt1, W2, b2, g2, bt2, W3, b3):
    emb_t = jnp.transpose(emb, (0, 2, 1))
    cat_t = lax.bitcast_convert_type(cat_data, jnp.float32).T
    xn_t = num_data.T

    xg_t = _gather(emb_t, cat_t)

    s1 = g1 * (1.0 / jnp.sqrt(1.0 + EPS))
    s2 = g2 * (1.0 / jnp.sqrt(1.0 + EPS))
    w1a = W1[:, : F * D]
    w1b = W1[:, F * D :]
    w2f = W2 * s1[None, :]
    b2f = W2 @ bt1 + b2
    w3f = W3 * s2[None, :]
    b3f = W3 @ bt2 + b3

    out_row = _mlp(
        xg_t,
        xn_t,
        w1a.astype(jnp.bfloat16),
        w1b.astype(jnp.bfloat16),
        b1.reshape(H, 1),
        w2f.astype(jnp.bfloat16),
        b2f.reshape(H, 1),
        w3f.astype(jnp.bfloat16),
        b3f.reshape(1, 1),
    )
    return out_row.reshape(B, 1)
